# baseline (speedup 1.0000x reference)
import functools

import numpy as np
import jax
import jax.numpy as jnp
from jax import lax
from jax.experimental import pallas as pl
from jax.experimental.pallas import tpu as pltpu
from jax.experimental.pallas import tpu_sc as plsc

MAX_LEN = 2048
MODEL_DIM = 768
BATCH = 4


def _build_pe(max_len, model_dim):
    pos = np.arange(max_len)[:, None]
    pe = pos / np.power(10000, 2.0 * np.arange(model_dim)[None, :] / model_dim)
    pe[:, 0::2] = np.sin(pe[:, 0::2])
    pe[:, 1::2] = np.cos(pe[:, 1::2])
    return pe.astype(np.float32).reshape(-1)


_PE = _build_pe(MAX_LEN, MODEL_DIM)

_info = plsc.get_sparse_core_info()
_NC, _NS, _L = _info.num_cores, _info.num_subcores, _info.num_lanes
_NW = _NC * _NS
_TPW = MAX_LEN // _NW
_VPR = MODEL_DIM // _L
_C = 32
_HPW = _TPW // _C
_NCHUNK = BATCH * _HPW
_NBUF = 3

_mesh = plsc.VectorSubcoreMesh(core_axis_name="c", subcore_axis_name="s")


@functools.partial(
    pl.kernel,
    mesh=_mesh,
    out_type=jax.ShapeDtypeStruct((BATCH * MAX_LEN, MODEL_DIM), jnp.float32),
    scratch_types=[
        pltpu.VMEM((BATCH, _TPW), jnp.int32),
        pltpu.VMEM((_TPW * MODEL_DIM,), jnp.float32),
        pltpu.VMEM((_NBUF, _C, MODEL_DIM), jnp.float32),
        pltpu.SemaphoreType.DMA,
        pltpu.SemaphoreType.DMA,
        pltpu.SemaphoreType.DMA,
        pltpu.SemaphoreType.DMA,
        pltpu.SemaphoreType.DMA,
        pltpu.SemaphoreType.DMA,
        pltpu.SemaphoreType.DMA,
        pltpu.SemaphoreType.DMA,
    ],
)
def _embed(x_hbm, table_hbm, pe_hbm, out_hbm, idx_v, pe_v, rows, *sems):
    gsems = sems[:_NBUF]
    wsems = sems[_NBUF:2 * _NBUF]
    isem, pesem = sems[2 * _NBUF], sems[2 * _NBUF + 1]
    wid = lax.axis_index("s") * _NC + lax.axis_index("c")
    t0 = wid * _TPW
    ih = pltpu.async_copy(
        x_hbm.at[pl.ds(t0, _TPW)], idx_v.at[0], isem)
    peh = pltpu.async_copy(
        pe_hbm.at[pl.ds(t0 * MODEL_DIM, _TPW * MODEL_DIM)], pe_v, pesem)
    gh = [None] * _NCHUNK
    wh = [None] * _NCHUNK
    waited = set()

    def start_gather(q):
        b, h = divmod(q, _HPW)
        gh[q] = pltpu.async_copy(
            table_hbm.at[idx_v.at[b, pl.ds(h * _C, _C)]],
            rows.at[q % _NBUF], gsems[q % _NBUF])

    ih.wait()
    for q in range(min(_NBUF - 1, _NCHUNK)):
        start_gather(q)
    for b in range(1, BATCH):
        pltpu.sync_copy(x_hbm.at[pl.ds(b * MAX_LEN + t0, _TPW)], idx_v.at[b])
    peh.wait()

    for q in range(_NCHUNK):
        b, h = divmod(q, _HPW)
        gh[q].wait()

        @plsc.parallel_loop(0, _C, unroll=4)
        def row_body(r, _k=q % _NBUF, _hb=h * _C):
            pe_base = (_hb + r) * MODEL_DIM
            for j in range(_VPR):
                plsc.addupdate(rows.at[_k, r, pl.ds(j * _L, _L)],
                               pe_v[pl.ds(pe_base + j * _L, _L)])

        wh[q] = pltpu.async_copy(
            rows.at[q % _NBUF],
            out_hbm.at[pl.ds(b * MAX_LEN + t0 + h * _C, _C)],
            wsems[q % _NBUF])

        if q + _NBUF - 1 < _NCHUNK:
            if q >= 1:
                wh[q - 1].wait()
                waited.add(q - 1)
            start_gather(q + _NBUF - 1)

    for q in range(_NCHUNK):
        if q not in waited:
            wh[q].wait()


def kernel(x, table):
    xf = x.reshape(-1).astype(jnp.int32)
    out = _embed(xf, table, jnp.asarray(_PE))
    return out.reshape(BATCH, MAX_LEN, MODEL_DIM)


# speedup vs baseline: 1.1555x; 1.1555x over previous
import functools

import numpy as np
import jax
import jax.numpy as jnp
from jax import lax
from jax.experimental import pallas as pl
from jax.experimental.pallas import tpu as pltpu
from jax.experimental.pallas import tpu_sc as plsc

MAX_LEN = 2048
MODEL_DIM = 768
BATCH = 4


def _build_pe(max_len, model_dim):
    pos = np.arange(max_len)[:, None]
    pe = pos / np.power(10000, 2.0 * np.arange(model_dim)[None, :] / model_dim)
    pe[:, 0::2] = np.sin(pe[:, 0::2])
    pe[:, 1::2] = np.cos(pe[:, 1::2])
    return pe.astype(np.float32).reshape(-1)


_PE = _build_pe(MAX_LEN, MODEL_DIM)

_info = plsc.get_sparse_core_info()
_NC, _NS, _L = _info.num_cores, _info.num_subcores, _info.num_lanes
_NW = _NC * _NS
_TPW = MAX_LEN // _NW
_VPR = MODEL_DIM // _L
_C = 32
_HPW = _TPW // _C
_NCHUNK = BATCH * _HPW
_NBUF = 3

_mesh = plsc.VectorSubcoreMesh(core_axis_name="c", subcore_axis_name="s")


@functools.partial(
    pl.kernel,
    mesh=_mesh,
    out_type=jax.ShapeDtypeStruct((BATCH * MAX_LEN, MODEL_DIM), jnp.float32),
    scratch_types=[
        pltpu.VMEM((BATCH, _TPW), jnp.int32),
        pltpu.VMEM((_TPW * MODEL_DIM,), jnp.float32),
        pltpu.VMEM((_NBUF, _C, MODEL_DIM), jnp.float32),
        pltpu.SemaphoreType.DMA,
        pltpu.SemaphoreType.DMA,
        pltpu.SemaphoreType.DMA,
        pltpu.SemaphoreType.DMA,
    ],
)
def _embed(x_hbm, table_hbm, pe_hbm, out_hbm, idx_v, pe_v, rows,
           gsem, wsem, isem, pesem):
    wid = lax.axis_index("s") * _NC + lax.axis_index("c")
    t0 = wid * _TPW

    ih = pltpu.async_copy(x_hbm.at[pl.ds(t0, _TPW)], idx_v.at[0], isem)
    peh = pltpu.async_copy(
        pe_hbm.at[pl.ds(t0 * MODEL_DIM, _TPW * MODEL_DIM)], pe_v, pesem)

    def start_gather(b, h, k):
        pltpu.async_copy(
            table_hbm.at[idx_v.at[b, pl.ds(h * _C, _C)]],
            rows.at[k], gsem)

    def wait_gather(k):
        pltpu.make_async_copy(
            table_hbm.at[pl.ds(0, _C)], rows.at[k], gsem).wait()

    def start_write(b, h, k):
        pltpu.async_copy(
            rows.at[k],
            out_hbm.at[pl.ds(b * MAX_LEN + t0 + h * _C, _C)], wsem)

    def wait_write():
        pltpu.make_async_copy(
            rows.at[0], out_hbm.at[pl.ds(0, _C)], wsem).wait()

    def add_pe(h, k):
        @plsc.parallel_loop(0, _C, unroll=4)
        def row_body(r):
            pe_base = (h * _C + r) * MODEL_DIM
            for j in range(_VPR):
                plsc.addupdate(rows.at[k, r, pl.ds(j * _L, _L)],
                               pe_v[pl.ds(pe_base + j * _L, _L)])

    ih.wait()
    start_gather(0, 0, 0)
    start_gather(0, 1, 1)
    for b in range(1, BATCH):
        pltpu.sync_copy(x_hbm.at[pl.ds(b * MAX_LEN + t0, _TPW)], idx_v.at[b])
    peh.wait()

    wait_gather(0)
    add_pe(0, 0)
    start_write(0, 0, 0)
    start_gather(1, 0, 2)

    def chunk_body(q, carry):
        b = q // _HPW
        h = lax.rem(q, _HPW)
        k = lax.rem(q, _NBUF)
        wait_gather(k)
        add_pe(h, k)
        start_write(b, h, k)
        wait_write()

        @pl.when(q + _NBUF - 1 < _NCHUNK)
        def _():
            q2 = q + _NBUF - 1
            start_gather(q2 // _HPW, lax.rem(q2, _HPW), lax.rem(q2, _NBUF))

        return carry

    lax.fori_loop(1, _NCHUNK, chunk_body, 0)
    wait_write()


def kernel(x, table):
    xf = x.reshape(-1).astype(jnp.int32)
    out = _embed(xf, table, jnp.asarray(_PE))
    return out.reshape(BATCH, MAX_LEN, MODEL_DIM)


# speedup vs baseline: 1.1991x; 1.0377x over previous
import functools

import numpy as np
import jax
import jax.numpy as jnp
from jax import lax
from jax.experimental import pallas as pl
from jax.experimental.pallas import tpu as pltpu
from jax.experimental.pallas import tpu_sc as plsc

MAX_LEN = 2048
MODEL_DIM = 768
BATCH = 4


def _build_pe(max_len, model_dim):
    pos = np.arange(max_len)[:, None]
    pe = pos / np.power(10000, 2.0 * np.arange(model_dim)[None, :] / model_dim)
    pe[:, 0::2] = np.sin(pe[:, 0::2])
    pe[:, 1::2] = np.cos(pe[:, 1::2])
    return pe.astype(np.float32).reshape(-1)


_PE = jax.device_put(_build_pe(MAX_LEN, MODEL_DIM))

_info = plsc.get_sparse_core_info()
_NC, _NS, _L = _info.num_cores, _info.num_subcores, _info.num_lanes
_NW = _NC * _NS
_TPW = MAX_LEN // _NW
_VPR = MODEL_DIM // _L
_C = 32
_HPW = _TPW // _C
_NCHUNK = BATCH * _HPW
_NBUF = 3

_mesh = plsc.VectorSubcoreMesh(core_axis_name="c", subcore_axis_name="s")


@functools.partial(
    pl.kernel,
    mesh=_mesh,
    out_type=jax.ShapeDtypeStruct((BATCH * MAX_LEN, MODEL_DIM), jnp.float32),
    scratch_types=[
        pltpu.VMEM((BATCH, _TPW), jnp.int32),
        pltpu.VMEM((_TPW * MODEL_DIM,), jnp.float32),
        pltpu.VMEM((_NBUF, _C, MODEL_DIM), jnp.float32),
        pltpu.SemaphoreType.DMA,
        pltpu.SemaphoreType.DMA,
        pltpu.SemaphoreType.DMA,
        pltpu.SemaphoreType.DMA,
    ],
)
def _embed(x_hbm, table_hbm, pe_hbm, out_hbm, idx_v, pe_v, rows,
           gsem, wsem, isem, pesem):
    wid = lax.axis_index("s") * _NC + lax.axis_index("c")
    t0 = wid * _TPW

    ih = pltpu.async_copy(x_hbm.at[0, pl.ds(t0, _TPW)], idx_v.at[0], isem)
    peh = pltpu.async_copy(
        pe_hbm.at[pl.ds(t0 * MODEL_DIM, _TPW * MODEL_DIM)], pe_v, pesem)

    def start_gather(b, h, k):
        pltpu.async_copy(
            table_hbm.at[idx_v.at[b, pl.ds(h * _C, _C)]],
            rows.at[k], gsem)

    def wait_gather(k):
        pltpu.make_async_copy(
            table_hbm.at[pl.ds(0, _C)], rows.at[k], gsem).wait()

    def start_write(b, h, k):
        pltpu.async_copy(
            rows.at[k],
            out_hbm.at[pl.ds(b * MAX_LEN + t0 + h * _C, _C)], wsem)

    def wait_write():
        pltpu.make_async_copy(
            rows.at[0], out_hbm.at[pl.ds(0, _C)], wsem).wait()

    def add_pe(h, k):
        @plsc.parallel_loop(0, _C, unroll=4)
        def row_body(r):
            pe_base = (h * _C + r) * MODEL_DIM
            for j in range(_VPR):
                plsc.addupdate(rows.at[k, r, pl.ds(j * _L, _L)],
                               pe_v[pl.ds(pe_base + j * _L, _L)])

    ih.wait()
    start_gather(0, 0, 0)
    start_gather(0, 1, 1)
    for b in range(1, BATCH):
        pltpu.sync_copy(x_hbm.at[b, pl.ds(t0, _TPW)], idx_v.at[b])
    peh.wait()

    wait_gather(0)
    add_pe(0, 0)
    start_write(0, 0, 0)
    start_gather(1, 0, 2)

    def chunk_body(q, carry):
        b = q // _HPW
        h = lax.rem(q, _HPW)
        k = lax.rem(q, _NBUF)
        wait_gather(k)
        add_pe(h, k)
        start_write(b, h, k)
        wait_write()

        @pl.when(q + _NBUF - 1 < _NCHUNK)
        def _():
            q2 = q + _NBUF - 1
            start_gather(q2 // _HPW, lax.rem(q2, _HPW), lax.rem(q2, _NBUF))

        return carry

    lax.fori_loop(1, _NCHUNK, chunk_body, 0)
    wait_write()


def kernel(x, table):
    xi = x.astype(jnp.int32)
    out = _embed(xi, table, _PE)
    return out.reshape(BATCH, MAX_LEN, MODEL_DIM)
